# baseline (speedup 1.0000x reference)
import functools

import jax
import jax.numpy as jnp
from jax import lax
from jax.experimental import pallas as pl
from jax.experimental.pallas import tpu as pltpu
from jax.experimental.pallas import tpu_sc as plsc

B, A, C = 32, 8732, 81
TAL = 1024
NA = (A + TAL - 1) // TAL
A2 = NA * TAL
LANES = 16
NCHUNK = A2 // LANES
_AUXW = 128
A4 = A * 4
NR4 = (A4 + 127) // 128
A4P = NR4 * 128
PAD = A2 - A


def _tc_body(cls_ref, tgt_ref, lp_ref, lt_ref, tgt4_ref, negce_ref,
             stats_ref):
    i = pl.program_id(1)

    @pl.when(i == 0)
    def _init():
        d = lp_ref[0] - lt_ref[0]
        ad = jnp.abs(d)
        sl1 = jnp.where(ad < 1.0, 0.5 * ad * ad, ad - 0.5)
        loc_blk = jnp.sum(jnp.where(tgt4_ref[0] > 0, sl1, 0.0))
        lane0 = lax.broadcasted_iota(jnp.int32, (1, 1, _AUXW), 2)
        stats_ref[...] = jnp.where(lane0 == 2, loc_blk, 0.0)

    x = cls_ref[0]
    tgt = jnp.reshape(tgt_ref[...], (1, TAL))
    a_idx = i * TAL + lax.broadcasted_iota(jnp.int32, (1, TAL), 1)
    valid = a_idx < A

    m = jnp.max(x, axis=0, keepdims=True)
    e = jnp.exp(x - m)
    lse = m + jnp.log(jnp.sum(e, axis=0, keepdims=True))
    tcl = jnp.clip(tgt, 0, C - 1)
    cio = lax.broadcasted_iota(jnp.int32, (C, TAL), 0)
    tlogit = jnp.sum(jnp.where(cio == tcl, x, 0.0), axis=0, keepdims=True)
    ce = jnp.where(valid, lse - tlogit, 0.0)

    pos = (tgt > 0) & valid
    negv = jnp.where(pos | ~valid, -1.0, jnp.maximum(ce, 0.0))
    negce_ref[...] = jnp.reshape(negv, (1, 1, 1, TAL))

    posf = jnp.where(pos, 1.0, 0.0)
    p_blk = jnp.sum(posf)
    posce_blk = jnp.sum(jnp.where(pos, ce, 0.0))

    lane = lax.broadcasted_iota(jnp.int32, (1, 1, _AUXW), 2)
    contrib = jnp.where(lane == 0, p_blk, 0.0)
    contrib = contrib + jnp.where(lane == 1, posce_blk, 0.0)
    stats_ref[...] = stats_ref[...] + contrib


def _tc_stage(loc_preds, loc_targets, cls_preds, cls_targets):
    bh = cls_preds.shape[0]
    cls_t = jnp.transpose(cls_preds, (0, 2, 1))
    tgt2 = jnp.pad(cls_targets, ((0, 0), (0, PAD))).reshape(bh, NA, 1, TAL)
    pad = ((0, 0), (0, A4P - A4))
    lp4 = jnp.pad(jnp.reshape(loc_preds, (bh, A4)), pad).reshape(bh, NR4, 128)
    lt4 = jnp.pad(jnp.reshape(loc_targets, (bh, A4)), pad).reshape(
        bh, NR4, 128)
    tgt4 = jnp.pad(
        jnp.repeat(cls_targets, 4, axis=1), pad).reshape(bh, NR4, 128)
    return pl.pallas_call(
        _tc_body,
        grid=(bh, NA),
        in_specs=[
            pl.BlockSpec((1, C, TAL), lambda b, i: (b, 0, i)),
            pl.BlockSpec((1, 1, 1, TAL), lambda b, i: (b, i, 0, 0)),
            pl.BlockSpec((1, NR4, 128), lambda b, i: (b, 0, 0)),
            pl.BlockSpec((1, NR4, 128), lambda b, i: (b, 0, 0)),
            pl.BlockSpec((1, NR4, 128), lambda b, i: (b, 0, 0)),
        ],
        out_specs=[
            pl.BlockSpec((1, 1, 1, TAL), lambda b, i: (b, i, 0, 0)),
            pl.BlockSpec((1, 1, _AUXW), lambda b, i: (b, 0, 0)),
        ],
        out_shape=[
            jax.ShapeDtypeStruct((bh, NA, 1, TAL), jnp.float32),
            jax.ShapeDtypeStruct((bh, 1, _AUXW), jnp.float32),
        ],
        compiler_params=pltpu.CompilerParams(
            dimension_semantics=("arbitrary", "arbitrary")),
    )(cls_t, tgt2, lp4, lt4, tgt4)


def _lane_sum(vec):
    s = vec[0]
    for j in range(1, LANES):
        s = s + vec[j]
    return s


def _sc_body(negce_hbm, out_hbm, row_v, res_v):
    wid = lax.axis_index("s") * 2 + lax.axis_index("c")
    pltpu.sync_copy(negce_hbm.at[wid], row_v)

    def pass1(i, carry):
        cnt, sm, mx = carry
        v = row_v[pl.ds(i * LANES, LANES)]
        isneg = v >= 0.0
        return (cnt + jnp.where(isneg, 1.0, 0.0),
                sm + jnp.where(isneg, v, 0.0),
                jnp.maximum(mx, jnp.where(isneg, v, 0.0)))

    cnt, sm, mx = lax.fori_loop(
        0, NCHUNK, pass1,
        (jnp.zeros((LANES,), jnp.float32), jnp.zeros((LANES,), jnp.float32),
         jnp.zeros((LANES,), jnp.float32)))
    n_neg = _lane_sum(cnt)
    sum_all = _lane_sum(sm)
    hi0 = mx[0]
    for j in range(1, LANES):
        hi0 = jnp.maximum(hi0, mx[j])
    p_row = jnp.float32(A) - n_neg

    def topk_sum(kf):
        def slow():
            def count_gt(t):
                def cpass(i, c):
                    v = row_v[pl.ds(i * LANES, LANES)]
                    return c + jnp.where(v > t, 1.0, 0.0)

                return _lane_sum(lax.fori_loop(
                    0, NCHUNK, cpass, jnp.zeros((LANES,), jnp.float32)))

            def bis_step(_, carry):
                lo, hi = carry
                mid = 0.5 * (lo + hi)
                c = count_gt(mid)
                lo2 = jnp.where(c >= kf, mid, lo)
                hi2 = jnp.where(c >= kf, hi, mid)
                return (lo2, hi2)

            lo, hi = lax.fori_loop(
                0, 40, bis_step, (jnp.float32(-1.0), hi0))

            def pass2(i, carry):
                cg, sg = carry
                v = row_v[pl.ds(i * LANES, LANES)]
                gt = v > hi
                return (cg + jnp.where(gt, 1.0, 0.0),
                        sg + jnp.where(gt, v, 0.0))

            cg, sg = lax.fori_loop(
                0, NCHUNK, pass2,
                (jnp.zeros((LANES,), jnp.float32),
                 jnp.zeros((LANES,), jnp.float32)))
            sum_gt = _lane_sum(sg)
            cnt_gt = _lane_sum(cg)
            return sum_gt + (kf - cnt_gt) * hi

        return lax.cond(
            kf < 0.5,
            lambda: jnp.float32(0.0),
            lambda: lax.cond(kf >= n_neg, lambda: sum_all, slow))

    extra3 = topk_sum(3.0 * p_row)
    extra5 = topk_sum(5.0 + p_row)

    res_v[pl.ds(0, LANES)] = jnp.full((LANES,), extra3, jnp.float32)
    res_v[pl.ds(LANES, LANES)] = jnp.full((LANES,), extra5, jnp.float32)
    pltpu.sync_copy(res_v, out_hbm.at[wid])


def _sc_stage(negce):
    mesh = plsc.VectorSubcoreMesh(core_axis_name="c", subcore_axis_name="s")
    return pl.kernel(
        _sc_body,
        out_type=jax.ShapeDtypeStruct((B, 2 * LANES), jnp.float32),
        mesh=mesh,
        scratch_types=[
            pltpu.VMEM((A2,), jnp.float32),
            pltpu.VMEM((2 * LANES,), jnp.float32),
        ],
    )(negce)


NSPLIT = 2
BH = B // NSPLIT


def kernel(loc_preds, loc_targets, cls_preds, cls_targets):
    negs, stats_l = [], []
    for h in range(NSPLIT):
        s = slice(h * BH, (h + 1) * BH)
        n4, st = _tc_stage(loc_preds[s], loc_targets[s], cls_preds[s],
                           cls_targets[s])
        negs.append(jnp.reshape(n4, (BH, A2)))
        stats_l.append(st)
    stats = jnp.concatenate(stats_l, axis=0)
    extras = _sc_stage(jnp.concatenate(negs, axis=0))

    p_r = stats[:, 0, 0]
    posce = jnp.sum(stats[:, 0, 1])
    locsum = jnp.sum(stats[:, 0, 2])
    npos = jnp.sum(p_r)
    haspos = npos > 0

    extra = jnp.sum(jnp.where(haspos, extras[:, 0], extras[:, LANES]))
    cls_sum = posce + extra
    k_r = jnp.where(haspos, 3.0 * p_r, 5.0 + p_r)
    total_neg = jnp.sum(jnp.minimum(k_r, float(A)))
    denom = jnp.where(haspos, npos, total_neg)
    cls_loss = cls_sum / denom
    loc_loss = jnp.where(haspos, locsum / npos, locsum)
    return loc_loss + cls_loss


# speedup vs baseline: 1.8199x; 1.8199x over previous
import functools

import jax
import jax.numpy as jnp
from jax import lax
from jax.experimental import pallas as pl
from jax.experimental.pallas import tpu as pltpu
from jax.experimental.pallas import tpu_sc as plsc

B, A, C = 32, 8732, 81
TAL = 2048
NA = (A + TAL - 1) // TAL
A2 = NA * TAL
LANES = 16
NCHUNK = A2 // LANES
_AUXW = 128
A4 = A * 4
NR4 = (A4 + 127) // 128
A4P = NR4 * 128
PAD = A2 - A


def _tc_body(cls_ref, tgt_ref, lp_ref, lt_ref, tgt4_ref, negce_ref,
             stats_ref):
    i = pl.program_id(1)

    @pl.when(i == 0)
    def _init():
        d = lp_ref[0] - lt_ref[0]
        ad = jnp.abs(d)
        sl1 = jnp.where(ad < 1.0, 0.5 * ad * ad, ad - 0.5)
        loc_blk = jnp.sum(jnp.where(tgt4_ref[0] > 0, sl1, 0.0))
        lane0 = lax.broadcasted_iota(jnp.int32, (1, 1, _AUXW), 2)
        stats_ref[...] = jnp.where(lane0 == 2, loc_blk, 0.0)

    x = cls_ref[0]
    tgt = jnp.reshape(tgt_ref[...], (1, TAL))
    a_idx = i * TAL + lax.broadcasted_iota(jnp.int32, (1, TAL), 1)
    valid = a_idx < A

    m = jnp.max(x, axis=0, keepdims=True)
    e = jnp.exp(x - m)
    lse = m + jnp.log(jnp.sum(e, axis=0, keepdims=True))
    tcl = jnp.clip(tgt, 0, C - 1)
    cio = lax.broadcasted_iota(jnp.int32, (C, TAL), 0)
    tlogit = jnp.sum(jnp.where(cio == tcl, x, 0.0), axis=0, keepdims=True)
    ce = jnp.where(valid, lse - tlogit, 0.0)

    pos = (tgt > 0) & valid
    negv = jnp.where(pos | ~valid, -1.0, jnp.maximum(ce, 0.0))
    negce_ref[...] = jnp.reshape(negv, (1, 1, 1, TAL))

    posf = jnp.where(pos, 1.0, 0.0)
    p_blk = jnp.sum(posf)
    posce_blk = jnp.sum(jnp.where(pos, ce, 0.0))

    lane = lax.broadcasted_iota(jnp.int32, (1, 1, _AUXW), 2)
    contrib = jnp.where(lane == 0, p_blk, 0.0)
    contrib = contrib + jnp.where(lane == 1, posce_blk, 0.0)
    stats_ref[...] = stats_ref[...] + contrib


def _tc_stage(loc_preds, loc_targets, cls_preds, cls_targets):
    bh = cls_preds.shape[0]
    cls_t = jnp.transpose(cls_preds, (0, 2, 1))
    tgt2 = jnp.pad(cls_targets, ((0, 0), (0, PAD))).reshape(bh, NA, 1, TAL)
    pad = ((0, 0), (0, A4P - A4))
    lp4 = jnp.pad(jnp.reshape(loc_preds, (bh, A4)), pad).reshape(bh, NR4, 128)
    lt4 = jnp.pad(jnp.reshape(loc_targets, (bh, A4)), pad).reshape(
        bh, NR4, 128)
    tgt4 = jnp.pad(
        jnp.repeat(cls_targets, 4, axis=1), pad).reshape(bh, NR4, 128)
    return pl.pallas_call(
        _tc_body,
        grid=(bh, NA),
        in_specs=[
            pl.BlockSpec((1, C, TAL), lambda b, i: (b, 0, i)),
            pl.BlockSpec((1, 1, 1, TAL), lambda b, i: (b, i, 0, 0)),
            pl.BlockSpec((1, NR4, 128), lambda b, i: (b, 0, 0)),
            pl.BlockSpec((1, NR4, 128), lambda b, i: (b, 0, 0)),
            pl.BlockSpec((1, NR4, 128), lambda b, i: (b, 0, 0)),
        ],
        out_specs=[
            pl.BlockSpec((1, 1, 1, TAL), lambda b, i: (b, i, 0, 0)),
            pl.BlockSpec((1, 1, _AUXW), lambda b, i: (b, 0, 0)),
        ],
        out_shape=[
            jax.ShapeDtypeStruct((bh, NA, 1, TAL), jnp.float32),
            jax.ShapeDtypeStruct((bh, 1, _AUXW), jnp.float32),
        ],
        compiler_params=pltpu.CompilerParams(
            dimension_semantics=("arbitrary", "arbitrary")),
    )(cls_t, tgt2, lp4, lt4, tgt4)


def _lane_sum(vec):
    s = vec[0]
    for j in range(1, LANES):
        s = s + vec[j]
    return s


def _sc_body(negce_hbm, out_hbm, row_v, res_v):
    wid = lax.axis_index("s") * 2 + lax.axis_index("c")
    pltpu.sync_copy(negce_hbm.at[wid], row_v)

    def pass1(i, carry):
        cnt, sm, mx = carry
        v = row_v[pl.ds(i * LANES, LANES)]
        isneg = v >= 0.0
        return (cnt + jnp.where(isneg, 1.0, 0.0),
                sm + jnp.where(isneg, v, 0.0),
                jnp.maximum(mx, jnp.where(isneg, v, 0.0)))

    cnt, sm, mx = lax.fori_loop(
        0, NCHUNK, pass1,
        (jnp.zeros((LANES,), jnp.float32), jnp.zeros((LANES,), jnp.float32),
         jnp.zeros((LANES,), jnp.float32)))
    n_neg = _lane_sum(cnt)
    sum_all = _lane_sum(sm)
    hi0 = mx[0]
    for j in range(1, LANES):
        hi0 = jnp.maximum(hi0, mx[j])
    p_row = jnp.float32(A) - n_neg

    def topk_sum(kf):
        def slow():
            def count_gt(t):
                def cpass(i, c):
                    v = row_v[pl.ds(i * LANES, LANES)]
                    return c + jnp.where(v > t, 1.0, 0.0)

                return _lane_sum(lax.fori_loop(
                    0, NCHUNK, cpass, jnp.zeros((LANES,), jnp.float32)))

            def bis_step(_, carry):
                lo, hi = carry
                mid = 0.5 * (lo + hi)
                c = count_gt(mid)
                lo2 = jnp.where(c >= kf, mid, lo)
                hi2 = jnp.where(c >= kf, hi, mid)
                return (lo2, hi2)

            lo, hi = lax.fori_loop(
                0, 40, bis_step, (jnp.float32(-1.0), hi0))

            def pass2(i, carry):
                cg, sg = carry
                v = row_v[pl.ds(i * LANES, LANES)]
                gt = v > hi
                return (cg + jnp.where(gt, 1.0, 0.0),
                        sg + jnp.where(gt, v, 0.0))

            cg, sg = lax.fori_loop(
                0, NCHUNK, pass2,
                (jnp.zeros((LANES,), jnp.float32),
                 jnp.zeros((LANES,), jnp.float32)))
            sum_gt = _lane_sum(sg)
            cnt_gt = _lane_sum(cg)
            return sum_gt + (kf - cnt_gt) * hi

        return lax.cond(
            kf < 0.5,
            lambda: jnp.float32(0.0),
            lambda: lax.cond(kf >= n_neg, lambda: sum_all, slow))

    extra3 = topk_sum(3.0 * p_row)
    extra5 = topk_sum(5.0 + p_row)

    res_v[pl.ds(0, LANES)] = jnp.full((LANES,), extra3, jnp.float32)
    res_v[pl.ds(LANES, LANES)] = jnp.full((LANES,), extra5, jnp.float32)
    pltpu.sync_copy(res_v, out_hbm.at[wid])


def _sc_stage(negce):
    mesh = plsc.VectorSubcoreMesh(core_axis_name="c", subcore_axis_name="s")
    return pl.kernel(
        _sc_body,
        out_type=jax.ShapeDtypeStruct((B, 2 * LANES), jnp.float32),
        mesh=mesh,
        scratch_types=[
            pltpu.VMEM((A2,), jnp.float32),
            pltpu.VMEM((2 * LANES,), jnp.float32),
        ],
    )(negce)


def kernel(loc_preds, loc_targets, cls_preds, cls_targets):
    negce4, stats = _tc_stage(loc_preds, loc_targets, cls_preds,
                              cls_targets)
    extras = _sc_stage(jnp.reshape(negce4, (B, A2)))

    p_r = stats[:, 0, 0]
    posce = jnp.sum(stats[:, 0, 1])
    locsum = jnp.sum(stats[:, 0, 2])
    npos = jnp.sum(p_r)
    haspos = npos > 0

    extra = jnp.sum(jnp.where(haspos, extras[:, 0], extras[:, LANES]))
    cls_sum = posce + extra
    k_r = jnp.where(haspos, 3.0 * p_r, 5.0 + p_r)
    total_neg = jnp.sum(jnp.minimum(k_r, float(A)))
    denom = jnp.where(haspos, npos, total_neg)
    cls_loss = cls_sum / denom
    loc_loss = jnp.where(haspos, locsum / npos, locsum)
    return loc_loss + cls_loss


# speedup vs baseline: 1.9338x; 1.0626x over previous
import functools

import jax
import jax.numpy as jnp
from jax import lax
from jax.experimental import pallas as pl
from jax.experimental.pallas import tpu as pltpu
from jax.experimental.pallas import tpu_sc as plsc

B, A, C = 32, 8732, 81
TAL = 4096
NA = (A + TAL - 1) // TAL
A2 = NA * TAL
LANES = 16
NCHUNK = A2 // LANES
_AUXW = 128
A4 = A * 4
NR4 = (A4 + 127) // 128
A4P = NR4 * 128
PAD = A2 - A


def _tc_body(cls_ref, tgt_ref, lp_ref, lt_ref, tgt4_ref, negce_ref,
             stats_ref):
    i = pl.program_id(1)

    @pl.when(i == 0)
    def _init():
        d = lp_ref[0] - lt_ref[0]
        ad = jnp.abs(d)
        sl1 = jnp.where(ad < 1.0, 0.5 * ad * ad, ad - 0.5)
        loc_blk = jnp.sum(jnp.where(tgt4_ref[0] > 0, sl1, 0.0))
        lane0 = lax.broadcasted_iota(jnp.int32, (1, 1, _AUXW), 2)
        stats_ref[...] = jnp.where(lane0 == 2, loc_blk, 0.0)

    x = cls_ref[0]
    tgt = jnp.reshape(tgt_ref[...], (1, TAL))
    a_idx = i * TAL + lax.broadcasted_iota(jnp.int32, (1, TAL), 1)
    valid = a_idx < A

    m = jnp.max(x, axis=0, keepdims=True)
    e = jnp.exp(x - m)
    lse = m + jnp.log(jnp.sum(e, axis=0, keepdims=True))
    tcl = jnp.clip(tgt, 0, C - 1)
    cio = lax.broadcasted_iota(jnp.int32, (C, TAL), 0)
    tlogit = jnp.sum(jnp.where(cio == tcl, x, 0.0), axis=0, keepdims=True)
    ce = jnp.where(valid, lse - tlogit, 0.0)

    pos = (tgt > 0) & valid
    negv = jnp.where(pos | ~valid, -1.0, jnp.maximum(ce, 0.0))
    negce_ref[...] = jnp.reshape(negv, (1, 1, 1, TAL))

    posf = jnp.where(pos, 1.0, 0.0)
    p_blk = jnp.sum(posf)
    posce_blk = jnp.sum(jnp.where(pos, ce, 0.0))

    lane = lax.broadcasted_iota(jnp.int32, (1, 1, _AUXW), 2)
    contrib = jnp.where(lane == 0, p_blk, 0.0)
    contrib = contrib + jnp.where(lane == 1, posce_blk, 0.0)
    stats_ref[...] = stats_ref[...] + contrib


def _tc_stage(loc_preds, loc_targets, cls_preds, cls_targets):
    bh = cls_preds.shape[0]
    cls_t = jnp.transpose(cls_preds, (0, 2, 1))
    tgt2 = jnp.pad(cls_targets, ((0, 0), (0, PAD))).reshape(bh, NA, 1, TAL)
    pad = ((0, 0), (0, A4P - A4))
    lp4 = jnp.pad(jnp.reshape(loc_preds, (bh, A4)), pad).reshape(bh, NR4, 128)
    lt4 = jnp.pad(jnp.reshape(loc_targets, (bh, A4)), pad).reshape(
        bh, NR4, 128)
    tgt4 = jnp.pad(
        jnp.repeat(cls_targets, 4, axis=1), pad).reshape(bh, NR4, 128)
    return pl.pallas_call(
        _tc_body,
        grid=(bh, NA),
        in_specs=[
            pl.BlockSpec((1, C, TAL), lambda b, i: (b, 0, i)),
            pl.BlockSpec((1, 1, 1, TAL), lambda b, i: (b, i, 0, 0)),
            pl.BlockSpec((1, NR4, 128), lambda b, i: (b, 0, 0)),
            pl.BlockSpec((1, NR4, 128), lambda b, i: (b, 0, 0)),
            pl.BlockSpec((1, NR4, 128), lambda b, i: (b, 0, 0)),
        ],
        out_specs=[
            pl.BlockSpec((1, 1, 1, TAL), lambda b, i: (b, i, 0, 0)),
            pl.BlockSpec((1, 1, _AUXW), lambda b, i: (b, 0, 0)),
        ],
        out_shape=[
            jax.ShapeDtypeStruct((bh, NA, 1, TAL), jnp.float32),
            jax.ShapeDtypeStruct((bh, 1, _AUXW), jnp.float32),
        ],
        compiler_params=pltpu.CompilerParams(
            dimension_semantics=("arbitrary", "arbitrary")),
    )(cls_t, tgt2, lp4, lt4, tgt4)


def _lane_sum(vec):
    s = vec[0]
    for j in range(1, LANES):
        s = s + vec[j]
    return s


def _sc_body(negce_hbm, out_hbm, row_v, res_v):
    wid = lax.axis_index("s") * 2 + lax.axis_index("c")
    pltpu.sync_copy(negce_hbm.at[wid], row_v)

    def pass1(i, carry):
        cnt, sm, mx = carry
        v = row_v[pl.ds(i * LANES, LANES)]
        isneg = v >= 0.0
        return (cnt + jnp.where(isneg, 1.0, 0.0),
                sm + jnp.where(isneg, v, 0.0),
                jnp.maximum(mx, jnp.where(isneg, v, 0.0)))

    cnt, sm, mx = lax.fori_loop(
        0, NCHUNK, pass1,
        (jnp.zeros((LANES,), jnp.float32), jnp.zeros((LANES,), jnp.float32),
         jnp.zeros((LANES,), jnp.float32)))
    n_neg = _lane_sum(cnt)
    sum_all = _lane_sum(sm)
    hi0 = mx[0]
    for j in range(1, LANES):
        hi0 = jnp.maximum(hi0, mx[j])
    p_row = jnp.float32(A) - n_neg

    def topk_sum(kf):
        def slow():
            def count_gt(t):
                def cpass(i, c):
                    v = row_v[pl.ds(i * LANES, LANES)]
                    return c + jnp.where(v > t, 1.0, 0.0)

                return _lane_sum(lax.fori_loop(
                    0, NCHUNK, cpass, jnp.zeros((LANES,), jnp.float32)))

            def bis_step(_, carry):
                lo, hi = carry
                mid = 0.5 * (lo + hi)
                c = count_gt(mid)
                lo2 = jnp.where(c >= kf, mid, lo)
                hi2 = jnp.where(c >= kf, hi, mid)
                return (lo2, hi2)

            lo, hi = lax.fori_loop(
                0, 40, bis_step, (jnp.float32(-1.0), hi0))

            def pass2(i, carry):
                cg, sg = carry
                v = row_v[pl.ds(i * LANES, LANES)]
                gt = v > hi
                return (cg + jnp.where(gt, 1.0, 0.0),
                        sg + jnp.where(gt, v, 0.0))

            cg, sg = lax.fori_loop(
                0, NCHUNK, pass2,
                (jnp.zeros((LANES,), jnp.float32),
                 jnp.zeros((LANES,), jnp.float32)))
            sum_gt = _lane_sum(sg)
            cnt_gt = _lane_sum(cg)
            return sum_gt + (kf - cnt_gt) * hi

        return lax.cond(
            kf < 0.5,
            lambda: jnp.float32(0.0),
            lambda: lax.cond(kf >= n_neg, lambda: sum_all, slow))

    extra3 = topk_sum(3.0 * p_row)
    extra5 = topk_sum(5.0 + p_row)

    res_v[pl.ds(0, LANES)] = jnp.full((LANES,), extra3, jnp.float32)
    res_v[pl.ds(LANES, LANES)] = jnp.full((LANES,), extra5, jnp.float32)
    pltpu.sync_copy(res_v, out_hbm.at[wid])


def _sc_stage(negce):
    mesh = plsc.VectorSubcoreMesh(core_axis_name="c", subcore_axis_name="s")
    return pl.kernel(
        _sc_body,
        out_type=jax.ShapeDtypeStruct((B, 2 * LANES), jnp.float32),
        mesh=mesh,
        scratch_types=[
            pltpu.VMEM((A2,), jnp.float32),
            pltpu.VMEM((2 * LANES,), jnp.float32),
        ],
    )(negce)


def kernel(loc_preds, loc_targets, cls_preds, cls_targets):
    negce4, stats = _tc_stage(loc_preds, loc_targets, cls_preds,
                              cls_targets)
    extras = _sc_stage(jnp.reshape(negce4, (B, A2)))

    p_r = stats[:, 0, 0]
    posce = jnp.sum(stats[:, 0, 1])
    locsum = jnp.sum(stats[:, 0, 2])
    npos = jnp.sum(p_r)
    haspos = npos > 0

    extra = jnp.sum(jnp.where(haspos, extras[:, 0], extras[:, LANES]))
    cls_sum = posce + extra
    k_r = jnp.where(haspos, 3.0 * p_r, 5.0 + p_r)
    total_neg = jnp.sum(jnp.minimum(k_r, float(A)))
    denom = jnp.where(haspos, npos, total_neg)
    cls_loss = cls_sum / denom
    loc_loss = jnp.where(haspos, locsum / npos, locsum)
    return loc_loss + cls_loss


# speedup vs baseline: 2.3999x; 1.2410x over previous
import functools

import jax
import jax.numpy as jnp
from jax import lax
from jax.experimental import pallas as pl
from jax.experimental.pallas import tpu as pltpu
from jax.experimental.pallas import tpu_sc as plsc

B, A, C = 32, 8732, 81
TAL = 8832
NA = (A + TAL - 1) // TAL
A2 = NA * TAL
LANES = 16
NCHUNK = A2 // LANES
_AUXW = 128
A4 = A * 4
NR4 = (A4 + 127) // 128
A4P = NR4 * 128
PAD = A2 - A


def _tc_body(cls_ref, tgt_ref, lp_ref, lt_ref, tgt4_ref, negce_ref,
             stats_ref):
    i = pl.program_id(1)

    @pl.when(i == 0)
    def _init():
        d = lp_ref[0] - lt_ref[0]
        ad = jnp.abs(d)
        sl1 = jnp.where(ad < 1.0, 0.5 * ad * ad, ad - 0.5)
        loc_blk = jnp.sum(jnp.where(tgt4_ref[0] > 0, sl1, 0.0))
        lane0 = lax.broadcasted_iota(jnp.int32, (1, 1, _AUXW), 2)
        stats_ref[...] = jnp.where(lane0 == 2, loc_blk, 0.0)

    x = cls_ref[0]
    tgt = jnp.reshape(tgt_ref[...], (1, TAL))
    a_idx = i * TAL + lax.broadcasted_iota(jnp.int32, (1, TAL), 1)
    valid = a_idx < A

    m = jnp.max(x, axis=0, keepdims=True)
    e = jnp.exp(x - m)
    lse = m + jnp.log(jnp.sum(e, axis=0, keepdims=True))
    tcl = jnp.clip(tgt, 0, C - 1)
    cio = lax.broadcasted_iota(jnp.int32, (C, TAL), 0)
    tlogit = jnp.sum(jnp.where(cio == tcl, x, 0.0), axis=0, keepdims=True)
    ce = jnp.where(valid, lse - tlogit, 0.0)

    pos = (tgt > 0) & valid
    negv = jnp.where(pos | ~valid, -1.0, jnp.maximum(ce, 0.0))
    negce_ref[...] = jnp.reshape(negv, (1, 1, 1, TAL))

    posf = jnp.where(pos, 1.0, 0.0)
    p_blk = jnp.sum(posf)
    posce_blk = jnp.sum(jnp.where(pos, ce, 0.0))

    lane = lax.broadcasted_iota(jnp.int32, (1, 1, _AUXW), 2)
    contrib = jnp.where(lane == 0, p_blk, 0.0)
    contrib = contrib + jnp.where(lane == 1, posce_blk, 0.0)
    stats_ref[...] = stats_ref[...] + contrib


def _tc_stage(loc_preds, loc_targets, cls_preds, cls_targets):
    bh = cls_preds.shape[0]
    cls_t = jnp.transpose(cls_preds, (0, 2, 1))
    tgt2 = jnp.pad(cls_targets, ((0, 0), (0, PAD))).reshape(bh, NA, 1, TAL)
    pad = ((0, 0), (0, A4P - A4))
    lp4 = jnp.pad(jnp.reshape(loc_preds, (bh, A4)), pad).reshape(bh, NR4, 128)
    lt4 = jnp.pad(jnp.reshape(loc_targets, (bh, A4)), pad).reshape(
        bh, NR4, 128)
    tgt4 = jnp.pad(
        jnp.repeat(cls_targets, 4, axis=1), pad).reshape(bh, NR4, 128)
    return pl.pallas_call(
        _tc_body,
        grid=(bh, NA),
        in_specs=[
            pl.BlockSpec((1, C, TAL), lambda b, i: (b, 0, i)),
            pl.BlockSpec((1, 1, 1, TAL), lambda b, i: (b, i, 0, 0)),
            pl.BlockSpec((1, NR4, 128), lambda b, i: (b, 0, 0)),
            pl.BlockSpec((1, NR4, 128), lambda b, i: (b, 0, 0)),
            pl.BlockSpec((1, NR4, 128), lambda b, i: (b, 0, 0)),
        ],
        out_specs=[
            pl.BlockSpec((1, 1, 1, TAL), lambda b, i: (b, i, 0, 0)),
            pl.BlockSpec((1, 1, _AUXW), lambda b, i: (b, 0, 0)),
        ],
        out_shape=[
            jax.ShapeDtypeStruct((bh, NA, 1, TAL), jnp.float32),
            jax.ShapeDtypeStruct((bh, 1, _AUXW), jnp.float32),
        ],
        compiler_params=pltpu.CompilerParams(
            dimension_semantics=("arbitrary", "arbitrary")),
    )(cls_t, tgt2, lp4, lt4, tgt4)


def _lane_sum(vec):
    s = vec[0]
    for j in range(1, LANES):
        s = s + vec[j]
    return s


def _sc_body(negce_hbm, out_hbm, row_v, res_v):
    wid = lax.axis_index("s") * 2 + lax.axis_index("c")
    pltpu.sync_copy(negce_hbm.at[wid], row_v)

    def pass1(i, carry):
        cnt, sm, mx = carry
        v = row_v[pl.ds(i * LANES, LANES)]
        isneg = v >= 0.0
        return (cnt + jnp.where(isneg, 1.0, 0.0),
                sm + jnp.where(isneg, v, 0.0),
                jnp.maximum(mx, jnp.where(isneg, v, 0.0)))

    cnt, sm, mx = lax.fori_loop(
        0, NCHUNK, pass1,
        (jnp.zeros((LANES,), jnp.float32), jnp.zeros((LANES,), jnp.float32),
         jnp.zeros((LANES,), jnp.float32)))
    n_neg = _lane_sum(cnt)
    sum_all = _lane_sum(sm)
    hi0 = mx[0]
    for j in range(1, LANES):
        hi0 = jnp.maximum(hi0, mx[j])
    p_row = jnp.float32(A) - n_neg

    def topk_sum(kf):
        def slow():
            def count_gt(t):
                def cpass(i, c):
                    v = row_v[pl.ds(i * LANES, LANES)]
                    return c + jnp.where(v > t, 1.0, 0.0)

                return _lane_sum(lax.fori_loop(
                    0, NCHUNK, cpass, jnp.zeros((LANES,), jnp.float32)))

            def bis_step(_, carry):
                lo, hi = carry
                mid = 0.5 * (lo + hi)
                c = count_gt(mid)
                lo2 = jnp.where(c >= kf, mid, lo)
                hi2 = jnp.where(c >= kf, hi, mid)
                return (lo2, hi2)

            lo, hi = lax.fori_loop(
                0, 40, bis_step, (jnp.float32(-1.0), hi0))

            def pass2(i, carry):
                cg, sg = carry
                v = row_v[pl.ds(i * LANES, LANES)]
                gt = v > hi
                return (cg + jnp.where(gt, 1.0, 0.0),
                        sg + jnp.where(gt, v, 0.0))

            cg, sg = lax.fori_loop(
                0, NCHUNK, pass2,
                (jnp.zeros((LANES,), jnp.float32),
                 jnp.zeros((LANES,), jnp.float32)))
            sum_gt = _lane_sum(sg)
            cnt_gt = _lane_sum(cg)
            return sum_gt + (kf - cnt_gt) * hi

        return lax.cond(
            kf < 0.5,
            lambda: jnp.float32(0.0),
            lambda: lax.cond(kf >= n_neg, lambda: sum_all, slow))

    extra3 = topk_sum(3.0 * p_row)
    extra5 = topk_sum(5.0 + p_row)

    res_v[pl.ds(0, LANES)] = jnp.full((LANES,), extra3, jnp.float32)
    res_v[pl.ds(LANES, LANES)] = jnp.full((LANES,), extra5, jnp.float32)
    pltpu.sync_copy(res_v, out_hbm.at[wid])


def _sc_stage(negce):
    mesh = plsc.VectorSubcoreMesh(core_axis_name="c", subcore_axis_name="s")
    return pl.kernel(
        _sc_body,
        out_type=jax.ShapeDtypeStruct((B, 2 * LANES), jnp.float32),
        mesh=mesh,
        scratch_types=[
            pltpu.VMEM((A2,), jnp.float32),
            pltpu.VMEM((2 * LANES,), jnp.float32),
        ],
    )(negce)


def kernel(loc_preds, loc_targets, cls_preds, cls_targets):
    negce4, stats = _tc_stage(loc_preds, loc_targets, cls_preds,
                              cls_targets)
    extras = _sc_stage(jnp.reshape(negce4, (B, A2)))

    p_r = stats[:, 0, 0]
    posce = jnp.sum(stats[:, 0, 1])
    locsum = jnp.sum(stats[:, 0, 2])
    npos = jnp.sum(p_r)
    haspos = npos > 0

    extra = jnp.sum(jnp.where(haspos, extras[:, 0], extras[:, LANES]))
    cls_sum = posce + extra
    k_r = jnp.where(haspos, 3.0 * p_r, 5.0 + p_r)
    total_neg = jnp.sum(jnp.minimum(k_r, float(A)))
    denom = jnp.where(haspos, npos, total_neg)
    cls_loss = cls_sum / denom
    loc_loss = jnp.where(haspos, locsum / npos, locsum)
    return loc_loss + cls_loss


# speedup vs baseline: 2.4003x; 1.0002x over previous
import functools

import jax
import jax.numpy as jnp
from jax import lax
from jax.experimental import pallas as pl
from jax.experimental.pallas import tpu as pltpu
from jax.experimental.pallas import tpu_sc as plsc

B, A, C = 32, 8732, 81
TAL = 8832
NA = (A + TAL - 1) // TAL
A2 = NA * TAL
LANES = 16
NCHUNK = A2 // LANES
_AUXW = 128
A4 = A * 4
NR4 = (A4 + 127) // 128
A4P = NR4 * 128
PAD = A2 - A


def _tc_body(cls_ref, tgt_ref, lp_ref, lt_ref, tgt4_ref, negce_ref,
             stats_ref):
    i = pl.program_id(1)

    @pl.when(i == 0)
    def _init():
        d = lp_ref[0] - lt_ref[0]
        ad = jnp.abs(d)
        sl1 = jnp.where(ad < 1.0, 0.5 * ad * ad, ad - 0.5)
        loc_blk = jnp.sum(jnp.where(tgt4_ref[0] > 0, sl1, 0.0))
        lane0 = lax.broadcasted_iota(jnp.int32, (1, 1, _AUXW), 2)
        stats_ref[...] = jnp.where(lane0 == 2, loc_blk, 0.0)

    x = cls_ref[0]
    tgt = jnp.reshape(tgt_ref[...], (1, TAL))
    a_idx = i * TAL + lax.broadcasted_iota(jnp.int32, (1, TAL), 1)
    valid = a_idx < A

    m = jnp.max(x, axis=0, keepdims=True)
    e = jnp.exp(x - m)
    lse = m + jnp.log(jnp.sum(e, axis=0, keepdims=True))
    tcl = jnp.clip(tgt, 0, C - 1)
    cio = lax.broadcasted_iota(jnp.int32, (C, TAL), 0)
    tlogit = jnp.sum(jnp.where(cio == tcl, x, 0.0), axis=0, keepdims=True)
    ce = jnp.where(valid, lse - tlogit, 0.0)

    pos = (tgt > 0) & valid
    negv = jnp.where(pos | ~valid, -1.0, jnp.maximum(ce, 0.0))
    negce_ref[...] = jnp.reshape(negv, (1, 1, 1, TAL))

    posf = jnp.where(pos, 1.0, 0.0)
    p_blk = jnp.sum(posf)
    posce_blk = jnp.sum(jnp.where(pos, ce, 0.0))

    lane = lax.broadcasted_iota(jnp.int32, (1, 1, _AUXW), 2)
    contrib = jnp.where(lane == 0, p_blk, 0.0)
    contrib = contrib + jnp.where(lane == 1, posce_blk, 0.0)
    stats_ref[...] = stats_ref[...] + contrib


def _tc_stage(loc_preds, loc_targets, cls_preds, cls_targets):
    bh = cls_preds.shape[0]
    cls_t = jnp.transpose(cls_preds, (0, 2, 1)) + 0.0
    tgt2 = jnp.pad(cls_targets, ((0, 0), (0, PAD))).reshape(bh, NA, 1, TAL)
    pad = ((0, 0), (0, A4P - A4))
    lp4 = jnp.pad(jnp.reshape(loc_preds, (bh, A4)), pad).reshape(bh, NR4, 128)
    lt4 = jnp.pad(jnp.reshape(loc_targets, (bh, A4)), pad).reshape(
        bh, NR4, 128)
    tgt4 = jnp.pad(
        jnp.repeat(cls_targets, 4, axis=1), pad).reshape(bh, NR4, 128)
    return pl.pallas_call(
        _tc_body,
        grid=(bh, NA),
        in_specs=[
            pl.BlockSpec((1, C, TAL), lambda b, i: (b, 0, i)),
            pl.BlockSpec((1, 1, 1, TAL), lambda b, i: (b, i, 0, 0)),
            pl.BlockSpec((1, NR4, 128), lambda b, i: (b, 0, 0)),
            pl.BlockSpec((1, NR4, 128), lambda b, i: (b, 0, 0)),
            pl.BlockSpec((1, NR4, 128), lambda b, i: (b, 0, 0)),
        ],
        out_specs=[
            pl.BlockSpec((1, 1, 1, TAL), lambda b, i: (b, i, 0, 0)),
            pl.BlockSpec((1, 1, _AUXW), lambda b, i: (b, 0, 0)),
        ],
        out_shape=[
            jax.ShapeDtypeStruct((bh, NA, 1, TAL), jnp.float32),
            jax.ShapeDtypeStruct((bh, 1, _AUXW), jnp.float32),
        ],
        compiler_params=pltpu.CompilerParams(
            dimension_semantics=("arbitrary", "arbitrary")),
    )(cls_t, tgt2, lp4, lt4, tgt4)


def _lane_sum(vec):
    s = vec[0]
    for j in range(1, LANES):
        s = s + vec[j]
    return s


def _sc_body(negce_hbm, out_hbm, row_v, res_v):
    wid = lax.axis_index("s") * 2 + lax.axis_index("c")
    pltpu.sync_copy(negce_hbm.at[wid], row_v)

    def pass1(i, carry):
        cnt, sm, mx = carry
        v = row_v[pl.ds(i * LANES, LANES)]
        isneg = v >= 0.0
        return (cnt + jnp.where(isneg, 1.0, 0.0),
                sm + jnp.where(isneg, v, 0.0),
                jnp.maximum(mx, jnp.where(isneg, v, 0.0)))

    cnt, sm, mx = lax.fori_loop(
        0, NCHUNK, pass1,
        (jnp.zeros((LANES,), jnp.float32), jnp.zeros((LANES,), jnp.float32),
         jnp.zeros((LANES,), jnp.float32)))
    n_neg = _lane_sum(cnt)
    sum_all = _lane_sum(sm)
    hi0 = mx[0]
    for j in range(1, LANES):
        hi0 = jnp.maximum(hi0, mx[j])
    p_row = jnp.float32(A) - n_neg

    def topk_sum(kf):
        def slow():
            def count_gt(t):
                def cpass(i, c):
                    v = row_v[pl.ds(i * LANES, LANES)]
                    return c + jnp.where(v > t, 1.0, 0.0)

                return _lane_sum(lax.fori_loop(
                    0, NCHUNK, cpass, jnp.zeros((LANES,), jnp.float32)))

            def bis_step(_, carry):
                lo, hi = carry
                mid = 0.5 * (lo + hi)
                c = count_gt(mid)
                lo2 = jnp.where(c >= kf, mid, lo)
                hi2 = jnp.where(c >= kf, hi, mid)
                return (lo2, hi2)

            lo, hi = lax.fori_loop(
                0, 40, bis_step, (jnp.float32(-1.0), hi0))

            def pass2(i, carry):
                cg, sg = carry
                v = row_v[pl.ds(i * LANES, LANES)]
                gt = v > hi
                return (cg + jnp.where(gt, 1.0, 0.0),
                        sg + jnp.where(gt, v, 0.0))

            cg, sg = lax.fori_loop(
                0, NCHUNK, pass2,
                (jnp.zeros((LANES,), jnp.float32),
                 jnp.zeros((LANES,), jnp.float32)))
            sum_gt = _lane_sum(sg)
            cnt_gt = _lane_sum(cg)
            return sum_gt + (kf - cnt_gt) * hi

        return lax.cond(
            kf < 0.5,
            lambda: jnp.float32(0.0),
            lambda: lax.cond(kf >= n_neg, lambda: sum_all, slow))

    extra3 = topk_sum(3.0 * p_row)
    extra5 = topk_sum(5.0 + p_row)

    res_v[pl.ds(0, LANES)] = jnp.full((LANES,), extra3, jnp.float32)
    res_v[pl.ds(LANES, LANES)] = jnp.full((LANES,), extra5, jnp.float32)
    pltpu.sync_copy(res_v, out_hbm.at[wid])


def _sc_stage(negce):
    mesh = plsc.VectorSubcoreMesh(core_axis_name="c", subcore_axis_name="s")
    return pl.kernel(
        _sc_body,
        out_type=jax.ShapeDtypeStruct((B, 2 * LANES), jnp.float32),
        mesh=mesh,
        scratch_types=[
            pltpu.VMEM((A2,), jnp.float32),
            pltpu.VMEM((2 * LANES,), jnp.float32),
        ],
    )(negce)


def kernel(loc_preds, loc_targets, cls_preds, cls_targets):
    negce4, stats = _tc_stage(loc_preds, loc_targets, cls_preds,
                              cls_targets)
    extras = _sc_stage(jnp.reshape(negce4, (B, A2)))

    p_r = stats[:, 0, 0]
    posce = jnp.sum(stats[:, 0, 1])
    locsum = jnp.sum(stats[:, 0, 2])
    npos = jnp.sum(p_r)
    haspos = npos > 0

    extra = jnp.sum(jnp.where(haspos, extras[:, 0], extras[:, LANES]))
    cls_sum = posce + extra
    k_r = jnp.where(haspos, 3.0 * p_r, 5.0 + p_r)
    total_neg = jnp.sum(jnp.minimum(k_r, float(A)))
    denom = jnp.where(haspos, npos, total_neg)
    cls_loss = cls_sum / denom
    loc_loss = jnp.where(haspos, locsum / npos, locsum)
    return loc_loss + cls_loss
